# baseline (speedup 1.0000x reference)
import functools

import jax
import jax.numpy as jnp
from jax import lax
from jax.experimental import pallas as pl
from jax.experimental.pallas import tpu as pltpu
from jax.experimental.pallas import tpu_sc as plsc

S = 2048
D = 1024
F = 2816
E = 8
BLK = 256
NB = S * 2 // BLK + E
ROWS = NB * BLK

NC = 2
NSUB = 16
NW = NC * NSUB
TPW = S // NW


def _router_body(x_ref, wr_ref, br_ref, pos0_ref, pos1_ref, p0_ref, p1_ref,
                 be_ref):
    x = x_ref[...]
    logits = jnp.dot(x, wr_ref[...],
                     preferred_element_type=jnp.float32) + br_ref[...]
    lanes = lax.broadcasted_iota(jnp.int32, (S, E), 1)
    m0 = jnp.max(logits, axis=1, keepdims=True)
    i0 = jnp.min(jnp.where(logits == m0, lanes, E), axis=1, keepdims=True)
    rest = jnp.where(lanes == i0, -jnp.inf, logits)
    m1 = jnp.max(rest, axis=1, keepdims=True)
    i1 = jnp.min(jnp.where(rest == m1, lanes, E), axis=1, keepdims=True)
    p0_ref[...] = jnp.broadcast_to(1.0 / (1.0 + jnp.exp(m1 - m0)), (S, 16))
    p1_ref[...] = jnp.broadcast_to(1.0 / (1.0 + jnp.exp(m0 - m1)), (S, 16))

    oh0 = (lanes == i0).astype(jnp.float32)
    oh1 = (lanes == i1).astype(jnp.float32)

    tr = lax.broadcasted_iota(jnp.int32, (BLK, BLK), 0)
    tc = lax.broadcasted_iota(jnp.int32, (BLK, BLK), 1)
    ltri = (tr >= tc).astype(jnp.bfloat16)

    def tiled_cumsum(oh):
        parts = []
        carry = jnp.zeros((1, E), jnp.float32)
        for t in range(S // BLK):
            seg = oh[t * BLK:(t + 1) * BLK, :].astype(jnp.bfloat16)
            local = jnp.dot(ltri, seg, preferred_element_type=jnp.float32)
            parts.append(local + carry)
            carry = carry + local[-1:, :]
        return jnp.concatenate(parts, axis=0)

    c0 = tiled_cumsum(oh0)
    c1 = tiled_cumsum(oh1)
    cx = (c0 - oh0) + (c1 - oh1)
    counts = c0[-1:, :] + c1[-1:, :]
    padded = ((counts.astype(jnp.int32) + BLK - 1) // BLK) * BLK
    padf = padded.astype(jnp.float32)
    erow = lax.broadcasted_iota(jnp.int32, (E, E), 0)
    ecol = lax.broadcasted_iota(jnp.int32, (E, E), 1)
    upper = (erow < ecol).astype(jnp.float32)
    poffx = jnp.dot(padf, upper,
                    preferred_element_type=jnp.float32)
    start_plus_rank = poffx + cx
    pos0_ref[...] = jnp.sum(oh0 * start_plus_rank, axis=1,
                            keepdims=True).astype(jnp.int32)
    pos1_ref[...] = (jnp.sum(oh1 * (start_plus_rank + oh0), axis=1,
                             keepdims=True)).astype(jnp.int32)
    cum_pad = poffx + padf
    bstart = (lax.broadcasted_iota(jnp.int32, (NB, E), 0) * BLK)
    be = jnp.sum((bstart.astype(jnp.float32) >= cum_pad),
                 axis=1, keepdims=True).astype(jnp.int32)
    be_ref[...] = jnp.minimum(be, E - 1)


def _router(x2, Wr, br):
    return pl.pallas_call(
        _router_body,
        out_shape=[
            jax.ShapeDtypeStruct((S, 1), jnp.int32),
            jax.ShapeDtypeStruct((S, 1), jnp.int32),
            jax.ShapeDtypeStruct((S, 16), jnp.float32),
            jax.ShapeDtypeStruct((S, 16), jnp.float32),
            jax.ShapeDtypeStruct((NB, 1), jnp.int32),
        ],
    )(x2, Wr, br.reshape(1, E))


@functools.cache
def _sc_combine():
    mesh = plsc.VectorSubcoreMesh(core_axis_name="c", subcore_axis_name="s")
    CCH = 32

    @functools.partial(
        pl.kernel,
        mesh=mesh,
        out_type=jax.ShapeDtypeStruct((S, D), jnp.float32),
        scratch_types=[
            pltpu.VMEM((TPW,), jnp.int32),
            pltpu.VMEM((TPW,), jnp.int32),
            pltpu.VMEM((TPW, 16), jnp.float32),
            pltpu.VMEM((TPW, 16), jnp.float32),
            pltpu.VMEM((CCH, D), jnp.float32),
            pltpu.VMEM((CCH, D), jnp.float32),
            pltpu.SemaphoreType.DMA,
            pltpu.SemaphoreType.DMA,
            pltpu.SemaphoreType.DMA,
        ],
    )
    def sc_combine(ys_hbm, pos0_hbm, pos1_hbm, p0_hbm, p1_hbm, out_hbm,
                   q0_v, q1_v, pv0, pv1, ra, rb, sem_a, sem_b, sem_w):
        wid = lax.axis_index("s") * NC + lax.axis_index("c")
        base = wid * TPW
        pltpu.sync_copy(pos0_hbm.at[pl.ds(base, TPW)], q0_v)
        pltpu.sync_copy(pos1_hbm.at[pl.ds(base, TPW)], q1_v)
        pltpu.sync_copy(p0_hbm.at[pl.ds(base, TPW)], pv0)
        pltpu.sync_copy(p1_hbm.at[pl.ds(base, TPW)], pv1)

        for c in range(TPW // CCH):
            cp_a = pltpu.async_copy(
                ys_hbm.at[q0_v.at[pl.ds(c * CCH, CCH)]], ra, sem_a)
            cp_b = pltpu.async_copy(
                ys_hbm.at[q1_v.at[pl.ds(c * CCH, CCH)]], rb, sem_b)
            cp_a.wait()
            cp_b.wait()

            def row(i, _):
                s0 = pv0[c * CCH + i]
                s1 = pv1[c * CCH + i]

                def vec(v, _):
                    sl = pl.ds(v * 16, 16)
                    ra[i, sl] = s0 * ra[i, sl] + s1 * rb[i, sl]
                    return 0
                return lax.fori_loop(0, D // 16, vec, 0)


            lax.fori_loop(0, CCH, row, 0)
            pltpu.sync_copy(ra, out_hbm.at[pl.ds(base + c * CCH, CCH)])

    return sc_combine


def _ffn_body(be_ref, ids_ref, xf_ref, w1_ref, w3_ref, w2_ref,
              out_ref, xg0, xg1, sem0, sem1):
    b = pl.program_id(0)
    xgs = (xg0, xg1)
    sems = (sem0, sem1)

    def issue(blk, xg, sem):
        def body(i, _):
            pltpu.make_async_copy(
                xf_ref.at[ids_ref[blk * BLK + i]], xg.at[i], sem).start()
            return 0
        lax.fori_loop(0, BLK, body, 0, unroll=8)

    def compute(xg):
        xb = xg[...].astype(jnp.bfloat16)
        h1 = jnp.dot(xb, w1_ref[0], preferred_element_type=jnp.float32)
        h3 = jnp.dot(xb, w3_ref[0], preferred_element_type=jnp.float32)
        h = (h1 * jax.nn.sigmoid(h1)) * h3
        y = jnp.dot(h.astype(jnp.bfloat16), w2_ref[0],
                    preferred_element_type=jnp.float32)
        out_ref[...] = y

    @pl.when(b == 0)
    def _():
        issue(0, xg0, sem0)

    for par in (0, 1):
        @pl.when(b % 2 == par)
        def _(par=par):
            @pl.when(b + 1 < NB)
            def _():
                issue(b + 1, xgs[1 - par], sems[1 - par])

            pltpu.make_async_copy(
                xf_ref.at[pl.ds(0, BLK)], xgs[par], sems[par]).wait()
            compute(xgs[par])


def _ffn(x2, sorted_ids, block_expert, W1b, W3b, W2b):
    grid_spec = pltpu.PrefetchScalarGridSpec(
        num_scalar_prefetch=2,
        grid=(NB,),
        in_specs=[
            pl.BlockSpec((S, D), lambda b, be, ids: (0, 0)),
            pl.BlockSpec((1, D, F), lambda b, be, ids: (be[b], 0, 0)),
            pl.BlockSpec((1, D, F), lambda b, be, ids: (be[b], 0, 0)),
            pl.BlockSpec((1, F, D), lambda b, be, ids: (be[b], 0, 0)),
        ],
        out_specs=pl.BlockSpec((BLK, D), lambda b, be, ids: (b, 0)),
        scratch_shapes=[
            pltpu.VMEM((BLK, D), jnp.float32),
            pltpu.VMEM((BLK, D), jnp.float32),
            pltpu.SemaphoreType.DMA,
            pltpu.SemaphoreType.DMA,
        ],
    )
    return pl.pallas_call(
        _ffn_body,
        grid_spec=grid_spec,
        out_shape=jax.ShapeDtypeStruct((ROWS, D), jnp.float32),
    )(block_expert, sorted_ids, x2, W1b, W3b, W2b)


def kernel(x, Wr, br, W1, W2, W3):
    x2 = x.reshape(S, D)
    pos0, pos1, p0, p1, be = _router(x2, Wr, br)
    pos0 = pos0.reshape(S)
    pos1 = pos1.reshape(S)
    pos_flat = jnp.stack([pos0, pos1], axis=1).reshape(-1)
    tok = jnp.arange(2 * S, dtype=jnp.int32) // 2
    sorted_ids = jnp.zeros((ROWS,), jnp.int32).at[pos_flat].set(tok)
    ys = _ffn(x2, sorted_ids, be.reshape(NB),
              W1.astype(jnp.bfloat16), W3.astype(jnp.bfloat16),
              W2.astype(jnp.bfloat16))
    out = _sc_combine()(ys, pos0, pos1, p0, p1)
    return out.reshape(1, S, D)
